# baseline (speedup 1.0000x reference)
import jax
import jax.numpy as jnp
from jax import lax
from jax.experimental import pallas as pl
from jax.experimental.pallas import tpu as pltpu
from jax.experimental.pallas import tpu_sc as plsc

_LANES = 128
_NSC = 2
_NSUB = 16
_NW = _NSC * _NSUB


def _cdiv(a, b):
    return (a + b - 1) // b


def _sc_mesh():
    return plsc.VectorSubcoreMesh(core_axis_name="c", subcore_axis_name="s")


def _sc_degree(dst2d, zeros16, ones16, n_pad, nch_w):
    rows_w = n_pad // _NSUB

    def body(dst_hbm, zeros_hbm, ones_hbm, out_hbm, didx_all, ones_v, acc,
             ssem):
        cid = lax.axis_index("c")
        sid = lax.axis_index("s")
        wid = cid * _NSUB + sid
        pltpu.sync_copy(zeros_hbm, acc.at[pl.ds(sid * rows_w, rows_w)])
        pltpu.sync_copy(ones_hbm, ones_v)
        pltpu.sync_copy(dst_hbm.at[pl.ds(wid * nch_w, nch_w)], didx_all)
        plsc.subcore_barrier()

        def fire(ci, _):
            pltpu.async_copy(ones_v, acc.at[didx_all.at[ci]], ssem, add=True)
            return ()

        def drain(ci, _):
            pltpu.make_async_copy(ones_v, acc.at[didx_all.at[ci]], ssem).wait()
            return ()

        lax.fori_loop(0, nch_w, fire, ())
        lax.fori_loop(0, nch_w, drain, ())
        plsc.subcore_barrier()
        pltpu.sync_copy(acc.at[pl.ds(sid * rows_w, rows_w)],
                        out_hbm.at[cid, pl.ds(sid * rows_w, rows_w)])

    fn = pl.kernel(
        body,
        out_type=jax.ShapeDtypeStruct((_NSC, n_pad, 16), jnp.float32),
        mesh=_sc_mesh(),
        compiler_params=pltpu.CompilerParams(use_tc_tiling_on_sc=False),
        scratch_types=[
            pltpu.VMEM((nch_w, _LANES), jnp.int32),
            pltpu.VMEM((_LANES, 16), jnp.float32),
            pltpu.VMEM_SHARED((n_pad, 16), jnp.float32),
            pltpu.SemaphoreType.DMA,
        ],
    )
    return fn(dst2d, zeros16, ones16)


def _sc_spmm(y, src2d, dst2d, zeros_f, n_pad, nch_w):
    f = y.shape[1]
    rows_w = n_pad // _NSUB

    npairs = nch_w // 2

    def body(y_hbm, src_hbm, dst_hbm, zeros_hbm, out_hbm,
             sidx_all, didx_all, rows0, rows1, acc,
             gsem0, gsem1, ssem0, ssem1):
        cid = lax.axis_index("c")
        sid = lax.axis_index("s")
        wid = cid * _NSUB + sid
        pltpu.sync_copy(zeros_hbm, acc.at[pl.ds(sid * rows_w, rows_w)])
        pltpu.sync_copy(src_hbm.at[pl.ds(wid * nch_w, nch_w)], sidx_all)
        pltpu.sync_copy(dst_hbm.at[pl.ds(wid * nch_w, nch_w)], didx_all)
        plsc.subcore_barrier()

        pltpu.async_copy(y_hbm.at[sidx_all.at[0]], rows0, gsem0)
        pltpu.async_copy(y_hbm.at[sidx_all.at[1]], rows1, gsem1)

        def step(i, _):
            c0 = 2 * i
            c1 = c0 + 1
            pltpu.make_async_copy(y_hbm.at[sidx_all.at[c0]], rows0, gsem0).wait()
            pltpu.async_copy(rows0, acc.at[didx_all.at[c0]], ssem0, add=True)
            pltpu.make_async_copy(y_hbm.at[sidx_all.at[c1]], rows1, gsem1).wait()
            pltpu.async_copy(rows1, acc.at[didx_all.at[c1]], ssem1, add=True)
            ca = jnp.minimum(c0 + 2, nch_w - 1)
            cb = jnp.minimum(c1 + 2, nch_w - 1)
            pltpu.make_async_copy(rows0, acc.at[didx_all.at[c0]], ssem0).wait()
            pltpu.async_copy(y_hbm.at[sidx_all.at[ca]], rows0, gsem0)
            pltpu.make_async_copy(rows1, acc.at[didx_all.at[c1]], ssem1).wait()
            pltpu.async_copy(y_hbm.at[sidx_all.at[cb]], rows1, gsem1)
            return ()

        lax.fori_loop(0, npairs, step, ())
        pltpu.make_async_copy(y_hbm.at[sidx_all.at[nch_w - 1]], rows0,
                              gsem0).wait()
        pltpu.make_async_copy(y_hbm.at[sidx_all.at[nch_w - 1]], rows1,
                              gsem1).wait()
        plsc.subcore_barrier()
        pltpu.sync_copy(acc.at[pl.ds(sid * rows_w, rows_w)],
                        out_hbm.at[cid, pl.ds(sid * rows_w, rows_w)])

    fn = pl.kernel(
        body,
        out_type=jax.ShapeDtypeStruct((_NSC, n_pad, f), jnp.float32),
        mesh=_sc_mesh(),
        compiler_params=pltpu.CompilerParams(use_tc_tiling_on_sc=False),
        scratch_types=[
            pltpu.VMEM((nch_w, _LANES), jnp.int32),
            pltpu.VMEM((nch_w, _LANES), jnp.int32),
            pltpu.VMEM((_LANES, f), jnp.float32),
            pltpu.VMEM((_LANES, f), jnp.float32),
            pltpu.VMEM_SHARED((n_pad, f), jnp.float32),
            pltpu.SemaphoreType.DMA,
            pltpu.SemaphoreType.DMA,
            pltpu.SemaphoreType.DMA,
            pltpu.SemaphoreType.DMA,
        ],
    )
    return fn(y, src2d, dst2d, zeros_f)


def _deg_from_partials(degp_ref, n):
    deg = (degp_ref[0] + degp_ref[1]).sum(axis=-1) * (1.0 / 16.0) + 1.0
    return deg[:n]


def _tc_scale_first(degp, x, W, n):
    def body(degp_ref, x_ref, w_ref, z_ref):
        dis = lax.rsqrt(_deg_from_partials(degp_ref, n))
        xw = jnp.dot(x_ref[...], w_ref[...],
                     preferred_element_type=jnp.float32)
        z_ref[...] = xw * dis[:, None]

    return pl.pallas_call(
        body,
        out_shape=jax.ShapeDtypeStruct((n, W.shape[1]), jnp.float32),
    )(degp, x, W)


def _tc_mid(degp, up, z, n):
    def body(degp_ref, up_ref, z_ref, v_ref):
        dinv = 1.0 / _deg_from_partials(degp_ref, n)
        s = up_ref[0, :n, :] + up_ref[1, :n, :] + z_ref[...]
        v_ref[...] = s * dinv[:, None]

    return pl.pallas_call(
        body,
        out_shape=jax.ShapeDtypeStruct(z.shape, jnp.float32),
    )(degp, up, z)


def _tc_final(degp, wp, v, b2d, n):
    def body(degp_ref, wp_ref, v_ref, b_ref, o_ref):
        dis = lax.rsqrt(_deg_from_partials(degp_ref, n))
        logits = (wp_ref[0, :n, :] + wp_ref[1, :n, :] + v_ref[...])
        logits = logits * dis[:, None] + b_ref[...]
        m = jnp.max(logits, axis=-1, keepdims=True)
        ex = jnp.exp(logits - m)
        lse = jnp.log(jnp.sum(ex, axis=-1, keepdims=True)) + m
        o_ref[...] = logits - lse

    return pl.pallas_call(
        body,
        out_shape=jax.ShapeDtypeStruct(v.shape, jnp.float32),
    )(degp, wp, v, b2d)


def kernel(x, edge_index, W, b):
    n = x.shape[0]
    c_out = W.shape[1]
    e = edge_index.shape[1]

    n_pad = _cdiv(n + 1, _LANES) * _LANES
    nch = _cdiv(e, _LANES * _NW * 8) * _NW * 8
    nch_w = nch // _NW
    ep = nch * _LANES

    src = edge_index[0]
    dst = edge_index[1]
    pad = ep - e
    pad_i = jnp.arange(pad, dtype=src.dtype)
    src_p = jnp.concatenate(
        [src, pad_i % jnp.asarray(n, src.dtype)]).reshape(nch, _LANES)
    dst_p = jnp.concatenate(
        [dst, n + pad_i % jnp.asarray(n_pad - n, dst.dtype)]
    ).reshape(nch, _LANES)

    rows_w = n_pad // _NSUB
    zeros16 = jnp.zeros((rows_w, 16), jnp.float32)
    zerosf = jnp.zeros((rows_w, c_out), jnp.float32)
    ones16 = jnp.ones((_LANES, 16), jnp.float32)

    degp = _sc_degree(dst_p, zeros16, ones16, n_pad, nch_w)
    z = _tc_scale_first(degp, x, W, n)
    up = _sc_spmm(z, src_p, dst_p, zerosf, n_pad, nch_w)
    v = _tc_mid(degp, up, z, n)
    wp = _sc_spmm(v, src_p, dst_p, zerosf, n_pad, nch_w)
    return _tc_final(degp, wp, v, b.reshape(1, -1), n)


# speedup vs baseline: 1.1404x; 1.1404x over previous
import jax
import jax.numpy as jnp
from jax import lax
from jax.experimental import pallas as pl
from jax.experimental.pallas import tpu as pltpu
from jax.experimental.pallas import tpu_sc as plsc

_LANES = 128
_NSC = 2
_NSUB = 16
_NW = _NSC * _NSUB


def _cdiv(a, b):
    return (a + b - 1) // b


def _sc_mesh():
    return plsc.VectorSubcoreMesh(core_axis_name="c", subcore_axis_name="s")


def _sc_degree(dst2d, zeros16, ones16, n_pad, nch_w):
    rows_w = n_pad // _NSUB

    def body(dst_hbm, zeros_hbm, ones_hbm, out_hbm, didx_all, ones_v, acc,
             ssem):
        cid = lax.axis_index("c")
        sid = lax.axis_index("s")
        wid = cid * _NSUB + sid
        pltpu.sync_copy(zeros_hbm, acc.at[pl.ds(sid * rows_w, rows_w)])
        pltpu.sync_copy(ones_hbm, ones_v)
        pltpu.sync_copy(dst_hbm.at[pl.ds(wid * nch_w, nch_w)], didx_all)
        plsc.subcore_barrier()

        def fire(ci, _):
            pltpu.async_copy(ones_v, acc.at[didx_all.at[ci]], ssem, add=True)
            return ()

        def drain(ci, _):
            pltpu.make_async_copy(ones_v, acc.at[didx_all.at[ci]], ssem).wait()
            return ()

        lax.fori_loop(0, nch_w, fire, ())
        lax.fori_loop(0, nch_w, drain, ())
        plsc.subcore_barrier()
        pltpu.sync_copy(acc.at[pl.ds(sid * rows_w, rows_w)],
                        out_hbm.at[cid, pl.ds(sid * rows_w, rows_w)])

    fn = pl.kernel(
        body,
        out_type=jax.ShapeDtypeStruct((_NSC, n_pad, 16), jnp.float32),
        mesh=_sc_mesh(),
        compiler_params=pltpu.CompilerParams(use_tc_tiling_on_sc=False),
        scratch_types=[
            pltpu.VMEM((nch_w, _LANES), jnp.int32),
            pltpu.VMEM((_LANES, 16), jnp.float32),
            pltpu.VMEM_SHARED((n_pad, 16), jnp.float32),
            pltpu.SemaphoreType.DMA,
        ],
    )
    return fn(dst2d, zeros16, ones16)


def _sc_spmm(y, src2d, dst2d, zeros_f, n_pad, nch_w):
    f = y.shape[1]
    rows_w = n_pad // _NSUB

    npairs = nch_w // 2

    def body(y_hbm, src_hbm, dst_hbm, zeros_hbm, out_hbm,
             sidx_all, didx_all, rows0, rows1, acc,
             gsem0, gsem1, ssem0, ssem1):
        cid = lax.axis_index("c")
        sid = lax.axis_index("s")
        wid = cid * _NSUB + sid
        pltpu.sync_copy(zeros_hbm, acc.at[pl.ds(sid * rows_w, rows_w)])
        pltpu.sync_copy(src_hbm.at[pl.ds(wid * nch_w, nch_w)], sidx_all)
        pltpu.sync_copy(dst_hbm.at[pl.ds(wid * nch_w, nch_w)], didx_all)
        plsc.subcore_barrier()

        pltpu.async_copy(y_hbm.at[sidx_all.at[0]], rows0, gsem0)

        def step(i, _):
            c0 = 2 * i
            c1 = c0 + 1
            pltpu.async_copy(y_hbm.at[sidx_all.at[c1]], rows1, gsem1)
            pltpu.make_async_copy(y_hbm.at[sidx_all.at[c0]], rows0, gsem0).wait()
            pltpu.sync_copy(rows0, acc.at[didx_all.at[c0]], add=True)
            cn = jnp.minimum(c0 + 2, nch_w - 1)
            pltpu.async_copy(y_hbm.at[sidx_all.at[cn]], rows0, gsem0)
            pltpu.make_async_copy(y_hbm.at[sidx_all.at[c1]], rows1, gsem1).wait()
            pltpu.sync_copy(rows1, acc.at[didx_all.at[c1]], add=True)
            return ()

        lax.fori_loop(0, npairs, step, ())
        pltpu.make_async_copy(y_hbm.at[sidx_all.at[nch_w - 1]], rows0,
                              gsem0).wait()
        plsc.subcore_barrier()
        pltpu.sync_copy(acc.at[pl.ds(sid * rows_w, rows_w)],
                        out_hbm.at[cid, pl.ds(sid * rows_w, rows_w)])

    fn = pl.kernel(
        body,
        out_type=jax.ShapeDtypeStruct((_NSC, n_pad, f), jnp.float32),
        mesh=_sc_mesh(),
        compiler_params=pltpu.CompilerParams(use_tc_tiling_on_sc=False),
        scratch_types=[
            pltpu.VMEM((nch_w, _LANES), jnp.int32),
            pltpu.VMEM((nch_w, _LANES), jnp.int32),
            pltpu.VMEM((_LANES, f), jnp.float32),
            pltpu.VMEM((_LANES, f), jnp.float32),
            pltpu.VMEM_SHARED((n_pad, f), jnp.float32),
            pltpu.SemaphoreType.DMA,
            pltpu.SemaphoreType.DMA,
            pltpu.SemaphoreType.DMA,
            pltpu.SemaphoreType.DMA,
        ],
    )
    return fn(y, src2d, dst2d, zeros_f)


def _deg_from_partials(degp_ref, n):
    deg = (degp_ref[0] + degp_ref[1]).sum(axis=-1) * (1.0 / 16.0) + 1.0
    return deg[:n]


def _tc_scale_first(degp, x, W, n):
    def body(degp_ref, x_ref, w_ref, z_ref):
        dis = lax.rsqrt(_deg_from_partials(degp_ref, n))
        xw = jnp.dot(x_ref[...], w_ref[...],
                     preferred_element_type=jnp.float32)
        z_ref[...] = xw * dis[:, None]

    return pl.pallas_call(
        body,
        out_shape=jax.ShapeDtypeStruct((n, W.shape[1]), jnp.float32),
    )(degp, x, W)


def _tc_mid(degp, up, z, n):
    def body(degp_ref, up_ref, z_ref, v_ref):
        dinv = 1.0 / _deg_from_partials(degp_ref, n)
        s = up_ref[0, :n, :] + up_ref[1, :n, :] + z_ref[...]
        v_ref[...] = s * dinv[:, None]

    return pl.pallas_call(
        body,
        out_shape=jax.ShapeDtypeStruct(z.shape, jnp.float32),
    )(degp, up, z)


def _tc_final(degp, wp, v, b2d, n):
    def body(degp_ref, wp_ref, v_ref, b_ref, o_ref):
        dis = lax.rsqrt(_deg_from_partials(degp_ref, n))
        logits = (wp_ref[0, :n, :] + wp_ref[1, :n, :] + v_ref[...])
        logits = logits * dis[:, None] + b_ref[...]
        m = jnp.max(logits, axis=-1, keepdims=True)
        ex = jnp.exp(logits - m)
        lse = jnp.log(jnp.sum(ex, axis=-1, keepdims=True)) + m
        o_ref[...] = logits - lse

    return pl.pallas_call(
        body,
        out_shape=jax.ShapeDtypeStruct(v.shape, jnp.float32),
    )(degp, wp, v, b2d)


def kernel(x, edge_index, W, b):
    n = x.shape[0]
    c_out = W.shape[1]
    e = edge_index.shape[1]

    n_pad = _cdiv(n + 1, _LANES) * _LANES
    nch = _cdiv(e, _LANES * _NW * 8) * _NW * 8
    nch_w = nch // _NW
    ep = nch * _LANES

    src = edge_index[0]
    dst = edge_index[1]
    pad = ep - e
    pad_i = jnp.arange(pad, dtype=src.dtype)
    src_p = jnp.concatenate(
        [src, pad_i % jnp.asarray(n, src.dtype)]).reshape(nch, _LANES)
    dst_p = jnp.concatenate(
        [dst, n + pad_i % jnp.asarray(n_pad - n, dst.dtype)]
    ).reshape(nch, _LANES)

    rows_w = n_pad // _NSUB
    zeros16 = jnp.zeros((rows_w, 16), jnp.float32)
    zerosf = jnp.zeros((rows_w, c_out), jnp.float32)
    ones16 = jnp.ones((_LANES, 16), jnp.float32)

    degp = _sc_degree(dst_p, zeros16, ones16, n_pad, nch_w)
    z = _tc_scale_first(degp, x, W, n)
    up = _sc_spmm(z, src_p, dst_p, zerosf, n_pad, nch_w)
    v = _tc_mid(degp, up, z, n)
    wp = _sc_spmm(v, src_p, dst_p, zerosf, n_pad, nch_w)
    return _tc_final(degp, wp, v, b.reshape(1, -1), n)


# speedup vs baseline: 1.1533x; 1.0114x over previous
import jax
import jax.numpy as jnp
from jax import lax
from jax.experimental import pallas as pl
from jax.experimental.pallas import tpu as pltpu
from jax.experimental.pallas import tpu_sc as plsc

_LANES = 128
_NSC = 2
_NSUB = 16
_NW = _NSC * _NSUB


def _cdiv(a, b):
    return (a + b - 1) // b


def _sc_mesh():
    return plsc.VectorSubcoreMesh(core_axis_name="c", subcore_axis_name="s")


def _sc_degree(dst2d, zeros16, ones16, n_pad, nch_w):
    rows_w = n_pad // _NSUB
    ngrp = rows_w // 16

    def body(dst_hbm, zeros_hbm, ones_hbm, out_hbm, didx_all, ones_v, cnt_v,
             deg_v, acc, ssem):
        cid = lax.axis_index("c")
        sid = lax.axis_index("s")
        wid = cid * _NSUB + sid
        pltpu.sync_copy(zeros_hbm, acc.at[pl.ds(sid * rows_w, rows_w)])
        pltpu.sync_copy(ones_hbm, ones_v)
        pltpu.sync_copy(dst_hbm.at[pl.ds(wid * nch_w, nch_w)], didx_all)
        plsc.subcore_barrier()

        def fire(ci, _):
            pltpu.async_copy(ones_v, acc.at[didx_all.at[ci]], ssem, add=True)
            return ()

        def drain(ci, _):
            pltpu.make_async_copy(ones_v, acc.at[didx_all.at[ci]], ssem).wait()
            return ()

        lax.fori_loop(0, nch_w, fire, ())
        lax.fori_loop(0, nch_w, drain, ())
        plsc.subcore_barrier()

        pltpu.sync_copy(acc.at[pl.ds(sid * rows_w, rows_w)], cnt_v)
        riota = lax.iota(jnp.int32, 16)
        zidx = jnp.zeros((16,), jnp.int32)

        def compress(g, _):
            vals = plsc.load_gather(cnt_v, [g * 16 + riota, zidx])
            deg_v[pl.ds(g * 16, 16)] = vals
            return ()

        lax.fori_loop(0, ngrp, compress, ())
        pltpu.sync_copy(deg_v, out_hbm.at[cid, pl.ds(sid * rows_w, rows_w)])

    fn = pl.kernel(
        body,
        out_type=jax.ShapeDtypeStruct((_NSC, n_pad), jnp.float32),
        mesh=_sc_mesh(),
        compiler_params=pltpu.CompilerParams(use_tc_tiling_on_sc=False,
                                             needs_layout_passes=False),
        scratch_types=[
            pltpu.VMEM((nch_w, _LANES), jnp.int32),
            pltpu.VMEM((_LANES, 16), jnp.float32),
            pltpu.VMEM((rows_w, 16), jnp.float32),
            pltpu.VMEM((rows_w,), jnp.float32),
            pltpu.VMEM_SHARED((n_pad, 16), jnp.float32),
            pltpu.SemaphoreType.DMA,
        ],
    )
    return fn(dst2d, zeros16, ones16)


def _sc_spmm(y, src2d, dst2d, zeros_f, n_pad, nch_w):
    f = y.shape[1]
    rows_w = n_pad // _NSUB
    npairs = nch_w // 2

    def body(y_hbm, src_hbm, dst_hbm, zeros_hbm, out_hbm,
             sidx_all, didx_all, rows0, rows1, acc,
             gsem0, gsem1):
        cid = lax.axis_index("c")
        sid = lax.axis_index("s")
        wid = cid * _NSUB + sid
        pltpu.sync_copy(zeros_hbm, acc.at[pl.ds(sid * rows_w, rows_w)])
        pltpu.sync_copy(src_hbm.at[pl.ds(wid * nch_w, nch_w)], sidx_all)
        pltpu.sync_copy(dst_hbm.at[pl.ds(wid * nch_w, nch_w)], didx_all)
        plsc.subcore_barrier()

        pltpu.async_copy(y_hbm.at[sidx_all.at[0]], rows0, gsem0)

        def step(i, _):
            c0 = 2 * i
            c1 = c0 + 1
            pltpu.async_copy(y_hbm.at[sidx_all.at[c1]], rows1, gsem1)
            pltpu.make_async_copy(y_hbm.at[sidx_all.at[c0]], rows0, gsem0).wait()
            pltpu.sync_copy(rows0, acc.at[didx_all.at[c0]], add=True)
            cn = jnp.minimum(c0 + 2, nch_w - 1)
            pltpu.async_copy(y_hbm.at[sidx_all.at[cn]], rows0, gsem0)
            pltpu.make_async_copy(y_hbm.at[sidx_all.at[c1]], rows1, gsem1).wait()
            pltpu.sync_copy(rows1, acc.at[didx_all.at[c1]], add=True)
            return ()

        lax.fori_loop(0, npairs, step, ())
        pltpu.make_async_copy(y_hbm.at[sidx_all.at[nch_w - 1]], rows0,
                              gsem0).wait()
        plsc.subcore_barrier()
        pltpu.sync_copy(acc.at[pl.ds(sid * rows_w, rows_w)],
                        out_hbm.at[cid, pl.ds(sid * rows_w, rows_w)])

    fn = pl.kernel(
        body,
        out_type=jax.ShapeDtypeStruct((_NSC, n_pad, f), jnp.float32),
        mesh=_sc_mesh(),
        compiler_params=pltpu.CompilerParams(use_tc_tiling_on_sc=False),
        scratch_types=[
            pltpu.VMEM((nch_w, _LANES), jnp.int32),
            pltpu.VMEM((nch_w, _LANES), jnp.int32),
            pltpu.VMEM((_LANES, f), jnp.float32),
            pltpu.VMEM((_LANES, f), jnp.float32),
            pltpu.VMEM_SHARED((n_pad, f), jnp.float32),
            pltpu.SemaphoreType.DMA,
            pltpu.SemaphoreType.DMA,
        ],
    )
    return fn(y, src2d, dst2d, zeros_f)


def _tc_scale_first(degv128, x, W, Ee, Eo, n):
    c_out = W.shape[1]
    zb = 64
    n2 = n // 2
    grid = _cdiv(n2, zb)

    def body(degv_ref, x_ref, w_ref, ee_ref, eo_ref, z_ref, s_ref):
        pid = pl.program_id(0)
        deg = degv_ref[0, pid] + degv_ref[1, pid] + 1.0
        dis = lax.rsqrt(deg)[None, :]
        A = ee_ref[...] * dis
        B = eo_ref[...] * dis
        ones = jnp.ones((2 * zb, c_out), jnp.float32)
        s_left = jnp.dot(A, ones, preferred_element_type=jnp.float32)
        s_right = jnp.dot(B, ones, preferred_element_type=jnp.float32)
        s_ref[...] = jnp.concatenate([s_left, s_right], axis=1)
        xw = jnp.dot(x_ref[...], w_ref[...], preferred_element_type=jnp.float32)
        ze = jnp.dot(ee_ref[...], xw, preferred_element_type=jnp.float32)
        zo = jnp.dot(eo_ref[...], xw, preferred_element_type=jnp.float32)
        z_ref[...] = jnp.concatenate([s_left * ze, s_right * zo], axis=1)

    out_shape = [jax.ShapeDtypeStruct((n2, 2 * c_out), jnp.float32),
                 jax.ShapeDtypeStruct((n2, 2 * c_out), jnp.float32)]
    return pl.pallas_call(
        body,
        grid=(grid,),
        in_specs=[
            pl.BlockSpec(degv128.shape, lambda b: (0, 0, 0)),
            pl.BlockSpec((2 * zb, x.shape[1]), lambda b: (b, 0)),
            pl.BlockSpec((x.shape[1], c_out), lambda b: (0, 0)),
            pl.BlockSpec((zb, 2 * zb), lambda b: (0, 0)),
            pl.BlockSpec((zb, 2 * zb), lambda b: (0, 0)),
        ],
        out_specs=[
            pl.BlockSpec((zb, 2 * c_out), lambda b: (b, 0)),
            pl.BlockSpec((zb, 2 * c_out), lambda b: (b, 0)),
        ],
        out_shape=out_shape,
    )(degv128, x, W, Ee, Eo)


def _tc_mid(up128, z128, scale128):
    n2 = z128.shape[0]

    def body(up_ref, z_ref, s_ref, v_ref):
        u = up_ref[0, :n2, :] + up_ref[1, :n2, :]
        sc = s_ref[...]
        v_ref[...] = (u + z_ref[...]) * (sc * sc)

    return pl.pallas_call(
        body,
        out_shape=jax.ShapeDtypeStruct(z128.shape, jnp.float32),
    )(up128, z128, scale128)


def _tc_final(wp128, v128, scale128, b2):
    n2 = v128.shape[0]
    c_out = v128.shape[1] // 2

    def body(wp_ref, v_ref, s_ref, b_ref, o_ref):
        w = wp_ref[0, :n2, :] + wp_ref[1, :n2, :]
        logits = (w + v_ref[...]) * s_ref[...] + b_ref[...]

        def lsm(l):
            m = jnp.max(l, axis=-1, keepdims=True)
            ex = jnp.exp(l - m)
            return l - (jnp.log(jnp.sum(ex, axis=-1, keepdims=True)) + m)

        o_ref[...] = jnp.concatenate(
            [lsm(logits[:, :c_out]), lsm(logits[:, c_out:])], axis=1)

    return pl.pallas_call(
        body,
        out_shape=jax.ShapeDtypeStruct(v128.shape, jnp.float32),
    )(wp128, v128, scale128, b2)


def kernel(x, edge_index, W, b):
    n = x.shape[0]
    c_out = W.shape[1]
    e = edge_index.shape[1]

    n_pad = _cdiv(n + 1, 8 * _LANES) * 8 * _LANES
    nch = _cdiv(e, _LANES * _NW * 8) * _NW * 8
    nch_w = nch // _NW
    ep = nch * _LANES

    src = edge_index[0]
    dst = edge_index[1]
    pad = ep - e
    pad_i = jnp.arange(pad, dtype=src.dtype)
    src_p = jnp.concatenate(
        [src, pad_i % jnp.asarray(n, src.dtype)]).reshape(nch, _LANES)
    dst_p = jnp.concatenate(
        [dst, n + pad_i % jnp.asarray(n_pad - n, dst.dtype)]
    ).reshape(nch, _LANES)

    rows_w = n_pad // _NSUB
    zeros16 = jnp.zeros((rows_w, 16), jnp.float32)
    zerosf = jnp.zeros((rows_w, c_out), jnp.float32)
    ones16 = jnp.ones((_LANES, 16), jnp.float32)
    zb = 64
    j_iota = jnp.arange(2 * zb, dtype=jnp.int32)[None, :]
    i_iota = jnp.arange(zb, dtype=jnp.int32)[:, None]
    Ee = (j_iota == 2 * i_iota).astype(jnp.float32)
    Eo = (j_iota == 2 * i_iota + 1).astype(jnp.float32)
    b2 = jnp.concatenate([b, b]).reshape(1, 2 * c_out)

    degv = _sc_degree(dst_p, zeros16, ones16, n_pad, nch_w)
    degv128 = degv.reshape(_NSC, n_pad // _LANES, _LANES)
    z128, scale128 = _tc_scale_first(degv128, x, W, Ee, Eo, n)
    up = _sc_spmm(z128.reshape(n, c_out), src_p, dst_p, zerosf, n_pad, nch_w)
    v128 = _tc_mid(up.reshape(_NSC, n_pad // 2, 2 * c_out), z128, scale128)
    wp = _sc_spmm(v128.reshape(n, c_out), src_p, dst_p, zerosf, n_pad, nch_w)
    out128 = _tc_final(wp.reshape(_NSC, n_pad // 2, 2 * c_out), v128,
                       scale128, b2)
    return out128.reshape(n, c_out)


# speedup vs baseline: 1.2359x; 1.0716x over previous
import jax
import jax.numpy as jnp
from jax import lax
from jax.experimental import pallas as pl
from jax.experimental.pallas import tpu as pltpu
from jax.experimental.pallas import tpu_sc as plsc

_LANES = 128
_NSC = 2
_NSUB = 16
_NW = _NSC * _NSUB


def _cdiv(a, b):
    return (a + b - 1) // b


def _sc_mesh():
    return plsc.VectorSubcoreMesh(core_axis_name="c", subcore_axis_name="s")


def _sc_degree(dst2d, zeros16, ones16, n_pad, nch_w):
    rows_w = n_pad // _NSUB
    ngrp = rows_w // 16
    npair_w = rows_w // 2

    def body(dst_hbm, zeros_hbm, ones_hbm, outv_hbm, oute_hbm,
             didx_all, ones_v, cnt_v, deg_v, dege_v, acc, ssem):
        cid = lax.axis_index("c")
        sid = lax.axis_index("s")
        wid = cid * _NSUB + sid
        pltpu.sync_copy(zeros_hbm, acc.at[pl.ds(sid * rows_w, rows_w)])
        pltpu.sync_copy(ones_hbm, ones_v)
        pltpu.sync_copy(dst_hbm.at[pl.ds(wid * nch_w, nch_w)], didx_all)
        plsc.subcore_barrier()

        def fire(ci, _):
            pltpu.async_copy(ones_v, acc.at[didx_all.at[ci]], ssem, add=True)
            return ()

        def drain(ci, _):
            pltpu.make_async_copy(ones_v, acc.at[didx_all.at[ci]], ssem).wait()
            return ()

        lax.fori_loop(0, nch_w, fire, ())
        lax.fori_loop(0, nch_w, drain, ())
        plsc.subcore_barrier()

        pltpu.sync_copy(acc.at[pl.ds(sid * rows_w, rows_w)], cnt_v)
        riota = lax.iota(jnp.int32, 16)
        zidx = jnp.zeros((16,), jnp.int32)

        def compress(g, _):
            vals = plsc.load_gather(cnt_v, [g * 16 + riota, zidx])
            deg_v[pl.ds(g * 16, 16)] = vals
            return ()

        lax.fori_loop(0, ngrp, compress, ())
        pltpu.sync_copy(deg_v, outv_hbm.at[cid, pl.ds(sid * rows_w, rows_w)])

        def expand(r, _):
            v0 = cnt_v[2 * r, :]
            v1 = cnt_v[2 * r + 1, :]
            for k in range(4):
                dege_v[r, pl.ds(16 * k, 16)] = v0
            for k in range(4, 8):
                dege_v[r, pl.ds(16 * k, 16)] = v1
            return ()

        lax.fori_loop(0, npair_w, expand, ())
        pltpu.sync_copy(dege_v, oute_hbm.at[cid, pl.ds(sid * npair_w, npair_w)])

    fn = pl.kernel(
        body,
        out_type=[jax.ShapeDtypeStruct((_NSC, n_pad), jnp.float32),
                  jax.ShapeDtypeStruct((_NSC, n_pad // 2, 128), jnp.float32)],
        mesh=_sc_mesh(),
        compiler_params=pltpu.CompilerParams(use_tc_tiling_on_sc=False,
                                             needs_layout_passes=False),
        scratch_types=[
            pltpu.VMEM((nch_w, _LANES), jnp.int32),
            pltpu.VMEM((_LANES, 16), jnp.float32),
            pltpu.VMEM((rows_w, 16), jnp.float32),
            pltpu.VMEM((rows_w,), jnp.float32),
            pltpu.VMEM((npair_w, 128), jnp.float32),
            pltpu.VMEM_SHARED((n_pad, 16), jnp.float32),
            pltpu.SemaphoreType.DMA,
        ],
    )
    return fn(dst2d, zeros16, ones16)


def _sc_spmm(y, src2d, dst2d, zeros_f, n_pad, nch_w):
    f = y.shape[1]
    rows_w = n_pad // _NSUB
    npairs = nch_w // 2

    def body(y_hbm, src_hbm, dst_hbm, zeros_hbm, out_hbm,
             sidx_all, didx_all, rows0, rows1, acc,
             gsem0, gsem1):
        cid = lax.axis_index("c")
        sid = lax.axis_index("s")
        wid = cid * _NSUB + sid
        pltpu.sync_copy(zeros_hbm, acc.at[pl.ds(sid * rows_w, rows_w)])
        pltpu.sync_copy(src_hbm.at[pl.ds(wid * nch_w, nch_w)], sidx_all)
        pltpu.sync_copy(dst_hbm.at[pl.ds(wid * nch_w, nch_w)], didx_all)
        plsc.subcore_barrier()

        pltpu.async_copy(y_hbm.at[sidx_all.at[0]], rows0, gsem0)

        def step(i, _):
            c0 = 2 * i
            c1 = c0 + 1
            pltpu.async_copy(y_hbm.at[sidx_all.at[c1]], rows1, gsem1)
            pltpu.make_async_copy(y_hbm.at[sidx_all.at[c0]], rows0, gsem0).wait()
            pltpu.sync_copy(rows0, acc.at[didx_all.at[c0]], add=True)
            cn = jnp.minimum(c0 + 2, nch_w - 1)
            pltpu.async_copy(y_hbm.at[sidx_all.at[cn]], rows0, gsem0)
            pltpu.make_async_copy(y_hbm.at[sidx_all.at[c1]], rows1, gsem1).wait()
            pltpu.sync_copy(rows1, acc.at[didx_all.at[c1]], add=True)
            return ()

        lax.fori_loop(0, npairs, step, ())
        pltpu.make_async_copy(y_hbm.at[sidx_all.at[nch_w - 1]], rows0,
                              gsem0).wait()
        plsc.subcore_barrier()
        pltpu.sync_copy(acc.at[pl.ds(sid * rows_w, rows_w)],
                        out_hbm.at[cid, pl.ds(sid * rows_w, rows_w)])

    fn = pl.kernel(
        body,
        out_type=jax.ShapeDtypeStruct((_NSC, n_pad, f), jnp.float32),
        mesh=_sc_mesh(),
        compiler_params=pltpu.CompilerParams(use_tc_tiling_on_sc=False),
        scratch_types=[
            pltpu.VMEM((nch_w, _LANES), jnp.int32),
            pltpu.VMEM((nch_w, _LANES), jnp.int32),
            pltpu.VMEM((_LANES, f), jnp.float32),
            pltpu.VMEM((_LANES, f), jnp.float32),
            pltpu.VMEM_SHARED((n_pad, f), jnp.float32),
            pltpu.SemaphoreType.DMA,
            pltpu.SemaphoreType.DMA,
        ],
    )
    return fn(y, src2d, dst2d, zeros_f)


def _tc_scale_first(degv128, x, W, n):
    c_out = W.shape[1]
    xb = 1024
    grid = _cdiv(n, xb)

    def body(degv_ref, x_ref, w_ref, z_ref):
        pid = pl.program_id(0)
        nrow = xb // 128
        deg = (degv_ref[0, pl.ds(nrow * pid, nrow), :]
               + degv_ref[1, pl.ds(nrow * pid, nrow), :])
        dis_t = lax.transpose(lax.rsqrt(deg), (1, 0))
        dcol = jnp.concatenate(
            [dis_t[:, k:k + 1] for k in range(nrow)], axis=0)
        xw = jnp.dot(x_ref[...], w_ref[...],
                     preferred_element_type=jnp.float32)
        z_ref[...] = jnp.concatenate(
            [xw * dcol, jnp.zeros((xb, 128 - c_out), jnp.float32)], axis=1)

    return pl.pallas_call(
        body,
        grid=(grid,),
        in_specs=[
            pl.BlockSpec(degv128.shape, lambda b: (0, 0, 0)),
            pl.BlockSpec((xb, x.shape[1]), lambda b: (b, 0)),
            pl.BlockSpec((x.shape[1], c_out), lambda b: (0, 0)),
        ],
        out_specs=pl.BlockSpec((xb, 128), lambda b: (b, 0)),
        out_shape=jax.ShapeDtypeStruct((n, 128), jnp.float32),
    )(degv128, x, W)


def _tc_mid(up128, degE, n2):
    def body(up_ref, de_ref, v_ref):
        u = up_ref[0, :n2, :] + up_ref[1, :n2, :]
        deg = de_ref[0, :n2, :] + de_ref[1, :n2, :]
        v_ref[...] = u / deg

    return pl.pallas_call(
        body,
        out_shape=jax.ShapeDtypeStruct((n2, 128), jnp.float32),
    )(up128, degE)


def _tc_final(wp128, degE, b2, n2):
    c_out = b2.shape[1] // 2

    def body(wp_ref, de_ref, b_ref, o_ref):
        w = wp_ref[0, :n2, :] + wp_ref[1, :n2, :]
        deg = de_ref[0, :n2, :] + de_ref[1, :n2, :]
        logits = w * lax.rsqrt(deg) + b_ref[...]

        def lsm(l):
            m = jnp.max(l, axis=-1, keepdims=True)
            ex = jnp.exp(l - m)
            return l - (jnp.log(jnp.sum(ex, axis=-1, keepdims=True)) + m)

        o_ref[...] = jnp.concatenate(
            [lsm(logits[:, :c_out]), lsm(logits[:, c_out:])], axis=1)

    return pl.pallas_call(
        body,
        out_shape=jax.ShapeDtypeStruct((n2, 2 * c_out), jnp.float32),
    )(wp128, degE, b2)


def kernel(x, edge_index, W, b):
    n = x.shape[0]
    c_out = W.shape[1]
    e = edge_index.shape[1]
    n2 = n // 2

    n_pad = _cdiv(n + 1, 8 * _LANES) * 8 * _LANES
    e_tot = e + n
    nch = _cdiv(e_tot, _LANES * _NW * 8) * _NW * 8
    nch_w = nch // _NW
    ep = nch * _LANES

    src = edge_index[0]
    dst = edge_index[1]
    loop = jnp.arange(n, dtype=src.dtype)
    pad = ep - e_tot
    pad_i = jnp.arange(pad, dtype=src.dtype)
    src_all = jnp.concatenate([src, loop, pad_i % jnp.asarray(n, src.dtype)])
    src_p = src_all.reshape(nch, _LANES)
    src2_p = (src_all * 2).reshape(nch, _LANES)
    dst_p = jnp.concatenate(
        [dst, loop, n + pad_i % jnp.asarray(n_pad - n, dst.dtype)]
    ).reshape(nch, _LANES)

    rows_w = n_pad // _NSUB
    zeros16 = jnp.zeros((rows_w, 16), jnp.float32)
    zerosf = jnp.zeros((rows_w, c_out), jnp.float32)
    ones16 = jnp.ones((_LANES, 16), jnp.float32)
    b2 = jnp.concatenate([b, b]).reshape(1, 2 * c_out)

    degv, degE = _sc_degree(dst_p, zeros16, ones16, n_pad, nch_w)
    degv128 = degv.reshape(_NSC, n_pad // _LANES, _LANES)
    zwide = _tc_scale_first(degv128, x, W, n)
    up = _sc_spmm(zwide.reshape(2 * n, c_out), src2_p, dst_p, zerosf,
                  n_pad, nch_w)
    v128 = _tc_mid(up.reshape(_NSC, n_pad // 2, 2 * c_out), degE, n2)
    wp = _sc_spmm(v128.reshape(n, c_out), src_p, dst_p, zerosf, n_pad, nch_w)
    out128 = _tc_final(wp.reshape(_NSC, n_pad // 2, 2 * c_out), degE, b2, n2)
    return out128.reshape(n, c_out)


# speedup vs baseline: 1.2370x; 1.0008x over previous
import jax
import jax.numpy as jnp
from jax import lax
from jax.experimental import pallas as pl
from jax.experimental.pallas import tpu as pltpu
from jax.experimental.pallas import tpu_sc as plsc

_LANES = 128
_NSC = 2
_NSUB = 16
_NW = _NSC * _NSUB


def _cdiv(a, b):
    return (a + b - 1) // b


def _sc_mesh():
    return plsc.VectorSubcoreMesh(core_axis_name="c", subcore_axis_name="s")


def _tc_edge_prep(edge_index, n, n_pad, ep):
    e = edge_index.shape[1]
    blk = 8192
    grid = ep // blk
    trash = n_pad - n

    def body(ei_ref, s_ref, s2_ref, d_ref):
        gi = pl.program_id(0) * blk + lax.broadcasted_iota(jnp.int32, (blk,), 0)
        in_e = gi < e
        s = jnp.where(in_e, ei_ref[0, :], gi % n)
        d = jnp.where(in_e, ei_ref[1, :], n + gi % trash)
        s_ref[...] = s
        s2_ref[...] = 2 * s
        d_ref[...] = d

    return pl.pallas_call(
        body,
        grid=(grid,),
        in_specs=[pl.BlockSpec((2, blk), lambda b: (0, b))],
        out_specs=[pl.BlockSpec((blk,), lambda b: (b,))] * 3,
        out_shape=[jax.ShapeDtypeStruct((ep,), jnp.int32)] * 3,
    )(edge_index)


def _sc_degree(dst2d, zeros16, ones16, n_pad, nch_w):
    rows_w = n_pad // _NSUB
    ngrp = rows_w // 16
    npair_w = rows_w // 2

    def body(dst_hbm, zeros_hbm, ones_hbm, outv_hbm, oute_hbm,
             didx_all, ones_v, cnt_v, deg_v, dege_v, acc, ssem):
        cid = lax.axis_index("c")
        sid = lax.axis_index("s")
        wid = cid * _NSUB + sid
        pltpu.sync_copy(zeros_hbm, acc.at[pl.ds(sid * rows_w, rows_w)])
        pltpu.sync_copy(ones_hbm, ones_v)
        pltpu.sync_copy(dst_hbm.at[pl.ds(wid * nch_w, nch_w)], didx_all)
        plsc.subcore_barrier()

        def fire(ci, _):
            pltpu.async_copy(ones_v, acc.at[didx_all.at[ci]], ssem, add=True)
            return ()

        def drain(ci, _):
            pltpu.make_async_copy(ones_v, acc.at[didx_all.at[ci]], ssem).wait()
            return ()

        lax.fori_loop(0, nch_w, fire, ())
        lax.fori_loop(0, nch_w, drain, ())
        plsc.subcore_barrier()

        pltpu.sync_copy(acc.at[pl.ds(sid * rows_w, rows_w)], cnt_v)
        riota = lax.iota(jnp.int32, 16)
        zidx = jnp.zeros((16,), jnp.int32)

        def compress(g, _):
            vals = plsc.load_gather(cnt_v, [g * 16 + riota, zidx])
            deg_v[pl.ds(g * 16, 16)] = vals
            return ()

        lax.fori_loop(0, ngrp, compress, ())
        pltpu.sync_copy(deg_v, outv_hbm.at[cid, pl.ds(sid * rows_w, rows_w)])

        def expand(r, _):
            v0 = cnt_v[2 * r, :]
            v1 = cnt_v[2 * r + 1, :]
            for k in range(4):
                dege_v[r, pl.ds(16 * k, 16)] = v0
            for k in range(4, 8):
                dege_v[r, pl.ds(16 * k, 16)] = v1
            return ()

        lax.fori_loop(0, npair_w, expand, ())
        pltpu.sync_copy(dege_v, oute_hbm.at[cid, pl.ds(sid * npair_w, npair_w)])

    fn = pl.kernel(
        body,
        out_type=[jax.ShapeDtypeStruct((_NSC, n_pad), jnp.float32),
                  jax.ShapeDtypeStruct((_NSC, n_pad // 2, 128), jnp.float32)],
        mesh=_sc_mesh(),
        compiler_params=pltpu.CompilerParams(use_tc_tiling_on_sc=False,
                                             needs_layout_passes=False),
        scratch_types=[
            pltpu.VMEM((nch_w, _LANES), jnp.int32),
            pltpu.VMEM((_LANES, 16), jnp.float32),
            pltpu.VMEM((rows_w, 16), jnp.float32),
            pltpu.VMEM((rows_w,), jnp.float32),
            pltpu.VMEM((npair_w, 128), jnp.float32),
            pltpu.VMEM_SHARED((n_pad, 16), jnp.float32),
            pltpu.SemaphoreType.DMA,
        ],
    )
    return fn(dst2d, zeros16, ones16)


def _sc_spmm(y, doubled_idx, src2d, dst2d, zeros_f, n, n_pad, nch_w):
    f = y.shape[1]
    rows_w = n_pad // _NSUB
    npairs = nch_w // 2
    nself = rows_w // _LANES
    full_tiles = n // rows_w
    rem = n % rows_w

    def body(y_hbm, src_hbm, dst_hbm, zeros_hbm, out_hbm,
             sidx_all, didx_all, sidx_self, didx_self, rows0, rows1, acc,
             gsem0, gsem1):
        cid = lax.axis_index("c")
        sid = lax.axis_index("s")
        wid = cid * _NSUB + sid

        if doubled_idx:
            pltpu.sync_copy(zeros_hbm, acc.at[pl.ds(sid * rows_w, rows_w)])
            riota = lax.iota(jnp.int32, 16)
            base_node = sid * rows_w
            for c in range(nself):
                for g in range(8):
                    nodes = base_node + (c * 128 + g * 16) + riota
                    didx_self[c, pl.ds(16 * g, 16)] = nodes
                    sidx_self[c, pl.ds(16 * g, 16)] = (
                        jnp.minimum(nodes, n - 1) * 2)
        else:
            @pl.when(sid < full_tiles)
            def _():
                pltpu.sync_copy(y_hbm.at[pl.ds(sid * rows_w, rows_w)],
                                acc.at[pl.ds(sid * rows_w, rows_w)])

            @pl.when(sid >= full_tiles)
            def _():
                if rem:
                    pltpu.sync_copy(y_hbm.at[pl.ds(sid * rows_w, rem)],
                                    acc.at[pl.ds(sid * rows_w, rem)])
                pltpu.sync_copy(
                    zeros_hbm.at[pl.ds(0, rows_w - rem)],
                    acc.at[pl.ds(sid * rows_w + rem, rows_w - rem)])

        pltpu.sync_copy(src_hbm.at[pl.ds(wid * nch_w, nch_w)], sidx_all)
        pltpu.sync_copy(dst_hbm.at[pl.ds(wid * nch_w, nch_w)], didx_all)
        plsc.subcore_barrier()

        pltpu.async_copy(y_hbm.at[sidx_all.at[0]], rows0, gsem0)

        def step(i, _):
            c0 = 2 * i
            c1 = c0 + 1
            pltpu.async_copy(y_hbm.at[sidx_all.at[c1]], rows1, gsem1)
            pltpu.make_async_copy(y_hbm.at[sidx_all.at[c0]], rows0, gsem0).wait()
            pltpu.sync_copy(rows0, acc.at[didx_all.at[c0]], add=True)
            cn = jnp.minimum(c0 + 2, nch_w - 1)
            pltpu.async_copy(y_hbm.at[sidx_all.at[cn]], rows0, gsem0)
            pltpu.make_async_copy(y_hbm.at[sidx_all.at[c1]], rows1, gsem1).wait()
            pltpu.sync_copy(rows1, acc.at[didx_all.at[c1]], add=True)
            return ()

        lax.fori_loop(0, npairs, step, ())
        pltpu.make_async_copy(y_hbm.at[sidx_all.at[nch_w - 1]], rows0,
                              gsem0).wait()
        if doubled_idx:
            pltpu.async_copy(y_hbm.at[sidx_self.at[0]], rows0, gsem0)
            for c in range(nself):
                buf, sem = (rows0, gsem0) if c % 2 == 0 else (rows1, gsem1)
                if c + 1 < nself:
                    nbuf, nsem = (rows1, gsem1) if c % 2 == 0 else (rows0, gsem0)
                    pltpu.async_copy(y_hbm.at[sidx_self.at[c + 1]], nbuf, nsem)
                pltpu.make_async_copy(y_hbm.at[sidx_self.at[c]], buf,
                                      sem).wait()
                pltpu.sync_copy(buf, acc.at[didx_self.at[c]], add=True)
        plsc.subcore_barrier()
        pltpu.sync_copy(acc.at[pl.ds(sid * rows_w, rows_w)],
                        out_hbm.at[cid, pl.ds(sid * rows_w, rows_w)])

    fn = pl.kernel(
        body,
        out_type=jax.ShapeDtypeStruct((_NSC, n_pad, f), jnp.float32),
        mesh=_sc_mesh(),
        compiler_params=pltpu.CompilerParams(use_tc_tiling_on_sc=False),
        scratch_types=[
            pltpu.VMEM((nch_w, _LANES), jnp.int32),
            pltpu.VMEM((nch_w, _LANES), jnp.int32),
            pltpu.VMEM((nself, _LANES), jnp.int32),
            pltpu.VMEM((nself, _LANES), jnp.int32),
            pltpu.VMEM((_LANES, f), jnp.float32),
            pltpu.VMEM((_LANES, f), jnp.float32),
            pltpu.VMEM_SHARED((n_pad, f), jnp.float32),
            pltpu.SemaphoreType.DMA,
            pltpu.SemaphoreType.DMA,
        ],
    )
    return fn(y, src2d, dst2d, zeros_f)


def _tc_scale_first(degv128, x, W, n):
    c_out = W.shape[1]
    xb = 1024
    grid = _cdiv(n, xb)

    def body(degv_ref, x_ref, w_ref, z_ref):
        pid = pl.program_id(0)
        nrow = xb // 128
        deg = (degv_ref[0, pl.ds(nrow * pid, nrow), :]
               + degv_ref[1, pl.ds(nrow * pid, nrow), :]) + 1.0
        dis_t = lax.transpose(lax.rsqrt(deg), (1, 0))
        dcol = jnp.concatenate(
            [dis_t[:, k:k + 1] for k in range(nrow)], axis=0)
        xw = jnp.dot(x_ref[...], w_ref[...],
                     preferred_element_type=jnp.float32)
        z_ref[...] = jnp.concatenate(
            [xw * dcol, jnp.zeros((xb, 128 - c_out), jnp.float32)], axis=1)

    return pl.pallas_call(
        body,
        grid=(grid,),
        in_specs=[
            pl.BlockSpec(degv128.shape, lambda b: (0, 0, 0)),
            pl.BlockSpec((xb, x.shape[1]), lambda b: (b, 0)),
            pl.BlockSpec((x.shape[1], c_out), lambda b: (0, 0)),
        ],
        out_specs=pl.BlockSpec((xb, 128), lambda b: (b, 0)),
        out_shape=jax.ShapeDtypeStruct((n, 128), jnp.float32),
    )(degv128, x, W)


def _tc_mid(up128, degE, n2):
    def body(up_ref, de_ref, v_ref):
        u = up_ref[0, :n2, :] + up_ref[1, :n2, :]
        deg = de_ref[0, :n2, :] + de_ref[1, :n2, :] + 1.0
        v_ref[...] = u / deg

    return pl.pallas_call(
        body,
        out_shape=jax.ShapeDtypeStruct((n2, 128), jnp.float32),
    )(up128, degE)


def _tc_final(wp128, degE, b2, n2):
    c_out = b2.shape[1] // 2

    def body(wp_ref, de_ref, b_ref, o_ref):
        w = wp_ref[0, :n2, :] + wp_ref[1, :n2, :]
        deg = de_ref[0, :n2, :] + de_ref[1, :n2, :] + 1.0
        logits = w * lax.rsqrt(deg) + b_ref[...]

        def lsm(l):
            m = jnp.max(l, axis=-1, keepdims=True)
            ex = jnp.exp(l - m)
            return l - (jnp.log(jnp.sum(ex, axis=-1, keepdims=True)) + m)

        o_ref[...] = jnp.concatenate(
            [lsm(logits[:, :c_out]), lsm(logits[:, c_out:])], axis=1)

    return pl.pallas_call(
        body,
        out_shape=jax.ShapeDtypeStruct((n2, 2 * c_out), jnp.float32),
    )(wp128, degE, b2)


def kernel(x, edge_index, W, b):
    n = x.shape[0]
    c_out = W.shape[1]
    e = edge_index.shape[1]
    n2 = n // 2

    n_pad = _cdiv(n + 1, 8 * _LANES) * 8 * _LANES
    nch = _cdiv(e, _LANES * _NW * 8) * _NW * 8
    nch_w = nch // _NW
    ep = nch * _LANES

    src_f, src2_f, dst_f = _tc_edge_prep(edge_index, n, n_pad, ep)
    src_p = src_f.reshape(nch, _LANES)
    src2_p = src2_f.reshape(nch, _LANES)
    dst_p = dst_f.reshape(nch, _LANES)

    rows_w = n_pad // _NSUB
    zeros16 = jnp.zeros((rows_w, 16), jnp.float32)
    zerosf = jnp.zeros((rows_w, c_out), jnp.float32)
    ones16 = jnp.ones((_LANES, 16), jnp.float32)
    b2 = jnp.concatenate([b, b]).reshape(1, 2 * c_out)

    degv, degE = _sc_degree(dst_p, zeros16, ones16, n_pad, nch_w)
    degv128 = degv.reshape(_NSC, n_pad // _LANES, _LANES)
    zwide = _tc_scale_first(degv128, x, W, n)
    up = _sc_spmm(zwide.reshape(2 * n, c_out), True,
                  src2_p, dst_p, zerosf, n, n_pad, nch_w)
    v128 = _tc_mid(up.reshape(_NSC, n_pad // 2, 2 * c_out), degE, n2)
    wp = _sc_spmm(v128.reshape(n, c_out), False,
                  src_p, dst_p, zerosf, n, n_pad, nch_w)
    out128 = _tc_final(wp.reshape(_NSC, n_pad // 2, 2 * c_out), degE, b2, n2)
    return out128.reshape(n, c_out)


# speedup vs baseline: 1.2572x; 1.0163x over previous
import jax
import jax.numpy as jnp
from jax import lax
from jax.experimental import pallas as pl
from jax.experimental.pallas import tpu as pltpu
from jax.experimental.pallas import tpu_sc as plsc

_LANES = 128
_NSC = 2
_NSUB = 16
_NW = _NSC * _NSUB


def _cdiv(a, b):
    return (a + b - 1) // b


def _sc_mesh():
    return plsc.VectorSubcoreMesh(core_axis_name="c", subcore_axis_name="s")


def _tc_edge_prep(edge_index, n, n_pad, ep):
    e = edge_index.shape[1]
    blk = 8192
    grid = ep // blk
    trash = n_pad - n

    def body(ei_ref, s_ref, s2_ref, d_ref):
        gi = pl.program_id(0) * blk + lax.broadcasted_iota(jnp.int32, (blk,), 0)
        in_e = gi < e
        s = jnp.where(in_e, ei_ref[0, :], gi % n)
        d = jnp.where(in_e, ei_ref[1, :], n + gi % trash)
        s_ref[...] = s
        s2_ref[...] = 2 * s
        d_ref[...] = d

    return pl.pallas_call(
        body,
        grid=(grid,),
        in_specs=[pl.BlockSpec((2, blk), lambda b: (0, b))],
        out_specs=[pl.BlockSpec((blk,), lambda b: (b,))] * 3,
        out_shape=[jax.ShapeDtypeStruct((ep,), jnp.int32)] * 3,
    )(edge_index)


def _sc_degree(dst2d, zeros16, ones16, n_pad, nch_w):
    rows_w = n_pad // _NSUB
    ngrp = rows_w // 16
    npair_w = rows_w // 2

    def body(dst_hbm, zeros_hbm, ones_hbm, outv_hbm, oute_hbm,
             didx_all, ones_v, cnt_v, deg_v, dege_v, acc, ssem):
        cid = lax.axis_index("c")
        sid = lax.axis_index("s")
        wid = cid * _NSUB + sid
        pltpu.sync_copy(zeros_hbm, acc.at[pl.ds(sid * rows_w, rows_w)])
        pltpu.sync_copy(ones_hbm, ones_v)
        pltpu.sync_copy(dst_hbm.at[pl.ds(wid * nch_w, nch_w)], didx_all)
        plsc.subcore_barrier()

        def fire(ci, _):
            pltpu.async_copy(ones_v, acc.at[didx_all.at[ci]], ssem, add=True)
            return ()

        def drain(ci, _):
            pltpu.make_async_copy(ones_v, acc.at[didx_all.at[ci]], ssem).wait()
            return ()

        lax.fori_loop(0, nch_w, fire, ())
        lax.fori_loop(0, nch_w, drain, ())
        plsc.subcore_barrier()

        pltpu.sync_copy(acc.at[pl.ds(sid * rows_w, rows_w)], cnt_v)
        riota = lax.iota(jnp.int32, 16)
        zidx = jnp.zeros((16,), jnp.int32)

        def compress(g, _):
            vals = plsc.load_gather(cnt_v, [g * 16 + riota, zidx])
            deg_v[pl.ds(g * 16, 16)] = vals
            return ()

        lax.fori_loop(0, ngrp, compress, ())
        pltpu.sync_copy(deg_v, outv_hbm.at[cid, pl.ds(sid * rows_w, rows_w)])

        def expand(r, _):
            v0 = cnt_v[2 * r, :]
            v1 = cnt_v[2 * r + 1, :]
            for k in range(4):
                dege_v[r, pl.ds(16 * k, 16)] = v0
            for k in range(4, 8):
                dege_v[r, pl.ds(16 * k, 16)] = v1
            return ()

        lax.fori_loop(0, npair_w, expand, ())
        pltpu.sync_copy(dege_v, oute_hbm.at[cid, pl.ds(sid * npair_w, npair_w)])

    fn = pl.kernel(
        body,
        out_type=[jax.ShapeDtypeStruct((_NSC, n_pad), jnp.float32),
                  jax.ShapeDtypeStruct((_NSC, n_pad // 2, 128), jnp.float32)],
        mesh=_sc_mesh(),
        compiler_params=pltpu.CompilerParams(use_tc_tiling_on_sc=False,
                                             needs_layout_passes=False),
        scratch_types=[
            pltpu.VMEM((nch_w, _LANES), jnp.int32),
            pltpu.VMEM((_LANES, 16), jnp.float32),
            pltpu.VMEM((rows_w, 16), jnp.float32),
            pltpu.VMEM((rows_w,), jnp.float32),
            pltpu.VMEM((npair_w, 128), jnp.float32),
            pltpu.VMEM_SHARED((n_pad, 16), jnp.float32),
            pltpu.SemaphoreType.DMA,
        ],
    )
    return fn(dst2d, zeros16, ones16)


def _sc_spmm(y, doubled_idx, src2d, dst2d, zeros_f, n, n_pad, nch_w):
    f = y.shape[1]
    rows_w = n_pad // _NSUB
    npairs = nch_w // 2
    nself = rows_w // _LANES
    full_tiles = n // rows_w
    rem = n % rows_w

    def body(y_hbm, src_hbm, dst_hbm, zeros_hbm, out_hbm,
             sidx_all, didx_all, sidx_self, didx_self, rows0, rows1, acc,
             gsem0, gsem1):
        cid = lax.axis_index("c")
        sid = lax.axis_index("s")
        wid = cid * _NSUB + sid

        if doubled_idx:
            pltpu.sync_copy(zeros_hbm, acc.at[pl.ds(sid * rows_w, rows_w)])
            riota = lax.iota(jnp.int32, 16)
            base_node = sid * rows_w
            for c in range(nself):
                for g in range(8):
                    nodes = base_node + (c * 128 + g * 16) + riota
                    didx_self[c, pl.ds(16 * g, 16)] = nodes
                    sidx_self[c, pl.ds(16 * g, 16)] = (
                        jnp.minimum(nodes, n - 1) * 2)
        else:
            @pl.when(jnp.logical_and(cid == 0, sid < full_tiles))
            def _():
                pltpu.sync_copy(y_hbm.at[pl.ds(sid * rows_w, rows_w)],
                                acc.at[pl.ds(sid * rows_w, rows_w)])

            @pl.when(jnp.logical_and(cid == 0, sid >= full_tiles))
            def _():
                if rem:
                    pltpu.sync_copy(y_hbm.at[pl.ds(sid * rows_w, rem)],
                                    acc.at[pl.ds(sid * rows_w, rem)])
                pltpu.sync_copy(
                    zeros_hbm.at[pl.ds(0, rows_w - rem)],
                    acc.at[pl.ds(sid * rows_w + rem, rows_w - rem)])

            @pl.when(cid != 0)
            def _():
                pltpu.sync_copy(zeros_hbm,
                                acc.at[pl.ds(sid * rows_w, rows_w)])

        pltpu.sync_copy(src_hbm.at[pl.ds(wid * nch_w, nch_w)], sidx_all)
        pltpu.sync_copy(dst_hbm.at[pl.ds(wid * nch_w, nch_w)], didx_all)
        plsc.subcore_barrier()

        pltpu.async_copy(y_hbm.at[sidx_all.at[0]], rows0, gsem0)

        def step(i, _):
            c0 = 2 * i
            c1 = c0 + 1
            pltpu.async_copy(y_hbm.at[sidx_all.at[c1]], rows1, gsem1)
            pltpu.make_async_copy(y_hbm.at[sidx_all.at[c0]], rows0, gsem0).wait()
            pltpu.sync_copy(rows0, acc.at[didx_all.at[c0]], add=True)
            cn = jnp.minimum(c0 + 2, nch_w - 1)
            pltpu.async_copy(y_hbm.at[sidx_all.at[cn]], rows0, gsem0)
            pltpu.make_async_copy(y_hbm.at[sidx_all.at[c1]], rows1, gsem1).wait()
            pltpu.sync_copy(rows1, acc.at[didx_all.at[c1]], add=True)
            return ()

        lax.fori_loop(0, npairs, step, ())
        pltpu.make_async_copy(y_hbm.at[sidx_all.at[nch_w - 1]], rows0,
                              gsem0).wait()
        if doubled_idx:
            @pl.when(cid == 0)
            def _():
                pltpu.async_copy(y_hbm.at[sidx_self.at[0]], rows0, gsem0)
                for c in range(nself):
                    buf, sem = (rows0, gsem0) if c % 2 == 0 else (rows1, gsem1)
                    if c + 1 < nself:
                        nbuf, nsem = ((rows1, gsem1) if c % 2 == 0
                                      else (rows0, gsem0))
                        pltpu.async_copy(y_hbm.at[sidx_self.at[c + 1]],
                                         nbuf, nsem)
                    pltpu.make_async_copy(y_hbm.at[sidx_self.at[c]], buf,
                                          sem).wait()
                    pltpu.sync_copy(buf, acc.at[didx_self.at[c]], add=True)
        plsc.subcore_barrier()
        pltpu.sync_copy(acc.at[pl.ds(sid * rows_w, rows_w)],
                        out_hbm.at[cid, pl.ds(sid * rows_w, rows_w)])

    fn = pl.kernel(
        body,
        out_type=jax.ShapeDtypeStruct((_NSC, n_pad, f), jnp.float32),
        mesh=_sc_mesh(),
        compiler_params=pltpu.CompilerParams(use_tc_tiling_on_sc=False),
        scratch_types=[
            pltpu.VMEM((nch_w, _LANES), jnp.int32),
            pltpu.VMEM((nch_w, _LANES), jnp.int32),
            pltpu.VMEM((nself, _LANES), jnp.int32),
            pltpu.VMEM((nself, _LANES), jnp.int32),
            pltpu.VMEM((_LANES, f), jnp.float32),
            pltpu.VMEM((_LANES, f), jnp.float32),
            pltpu.VMEM_SHARED((n_pad, f), jnp.float32),
            pltpu.SemaphoreType.DMA,
            pltpu.SemaphoreType.DMA,
        ],
    )
    return fn(y, src2d, dst2d, zeros_f)


def _tc_scale_first(degv128, x, W, n):
    c_out = W.shape[1]
    xb = 1024
    grid = _cdiv(n, xb)

    def body(degv_ref, x_ref, w_ref, z_ref):
        pid = pl.program_id(0)
        nrow = xb // 128
        deg = (degv_ref[0, pl.ds(nrow * pid, nrow), :]
               + degv_ref[1, pl.ds(nrow * pid, nrow), :]) + 1.0
        dis_t = lax.transpose(lax.rsqrt(deg), (1, 0))
        dcol = jnp.concatenate(
            [dis_t[:, k:k + 1] for k in range(nrow)], axis=0)
        xw = jnp.dot(x_ref[...], w_ref[...],
                     preferred_element_type=jnp.float32)
        z_ref[...] = jnp.concatenate(
            [xw * dcol, jnp.zeros((xb, 128 - c_out), jnp.float32)], axis=1)

    return pl.pallas_call(
        body,
        grid=(grid,),
        in_specs=[
            pl.BlockSpec(degv128.shape, lambda b: (0, 0, 0)),
            pl.BlockSpec((xb, x.shape[1]), lambda b: (b, 0)),
            pl.BlockSpec((x.shape[1], c_out), lambda b: (0, 0)),
        ],
        out_specs=pl.BlockSpec((xb, 128), lambda b: (b, 0)),
        out_shape=jax.ShapeDtypeStruct((n, 128), jnp.float32),
    )(degv128, x, W)


def _tc_mid(up128, degE, n2):
    def body(up_ref, de_ref, v_ref):
        u = up_ref[0, :n2, :] + up_ref[1, :n2, :]
        deg = de_ref[0, :n2, :] + de_ref[1, :n2, :] + 1.0
        v_ref[...] = u / deg

    return pl.pallas_call(
        body,
        out_shape=jax.ShapeDtypeStruct((n2, 128), jnp.float32),
    )(up128, degE)


def _tc_final(wp128, degE, b2, n2):
    c_out = b2.shape[1] // 2

    def body(wp_ref, de_ref, b_ref, o_ref):
        w = wp_ref[0, :n2, :] + wp_ref[1, :n2, :]
        deg = de_ref[0, :n2, :] + de_ref[1, :n2, :] + 1.0
        logits = w * lax.rsqrt(deg) + b_ref[...]

        def lsm(l):
            m = jnp.max(l, axis=-1, keepdims=True)
            ex = jnp.exp(l - m)
            return l - (jnp.log(jnp.sum(ex, axis=-1, keepdims=True)) + m)

        o_ref[...] = jnp.concatenate(
            [lsm(logits[:, :c_out]), lsm(logits[:, c_out:])], axis=1)

    return pl.pallas_call(
        body,
        out_shape=jax.ShapeDtypeStruct((n2, 2 * c_out), jnp.float32),
    )(wp128, degE, b2)


def kernel(x, edge_index, W, b):
    n = x.shape[0]
    c_out = W.shape[1]
    e = edge_index.shape[1]
    n2 = n // 2

    n_pad = _cdiv(n + 1, 8 * _LANES) * 8 * _LANES
    nch = _cdiv(e, _LANES * _NW * 8) * _NW * 8
    nch_w = nch // _NW
    ep = nch * _LANES

    src_f, src2_f, dst_f = _tc_edge_prep(edge_index, n, n_pad, ep)
    src_p = src_f.reshape(nch, _LANES)
    src2_p = src2_f.reshape(nch, _LANES)
    dst_p = dst_f.reshape(nch, _LANES)

    rows_w = n_pad // _NSUB
    zeros16 = jnp.zeros((rows_w, 16), jnp.float32)
    zerosf = jnp.zeros((rows_w, c_out), jnp.float32)
    ones16 = jnp.ones((_LANES, 16), jnp.float32)
    b2 = jnp.concatenate([b, b]).reshape(1, 2 * c_out)

    degv, degE = _sc_degree(dst_p, zeros16, ones16, n_pad, nch_w)
    degv128 = degv.reshape(_NSC, n_pad // _LANES, _LANES)
    zwide = _tc_scale_first(degv128, x, W, n)
    up = _sc_spmm(zwide.reshape(2 * n, c_out), True,
                  src2_p, dst_p, zerosf, n, n_pad, nch_w)
    v128 = _tc_mid(up.reshape(_NSC, n_pad // 2, 2 * c_out), degE, n2)
    wp = _sc_spmm(v128.reshape(n, c_out), False,
                  src_p, dst_p, zerosf, n, n_pad, nch_w)
    out128 = _tc_final(wp.reshape(_NSC, n_pad // 2, 2 * c_out), degE, b2, n2)
    return out128.reshape(n, c_out)


# speedup vs baseline: 1.3551x; 1.0779x over previous
import jax
import jax.numpy as jnp
from jax import lax
from jax.experimental import pallas as pl
from jax.experimental.pallas import tpu as pltpu
from jax.experimental.pallas import tpu_sc as plsc

_LANES = 128
_NSC = 2
_NSUB = 16
_NW = _NSC * _NSUB


def _cdiv(a, b):
    return (a + b - 1) // b


def _sc_mesh():
    return plsc.VectorSubcoreMesh(core_axis_name="c", subcore_axis_name="s")


def _tc_edge_prep_dst(edge_index, n, n_pad, ep):
    e = edge_index.shape[1]
    blk = 32768
    grid = ep // blk
    trash = n_pad - n

    def body(ei_ref, d_ref):
        gi = pl.program_id(0) * blk + lax.broadcasted_iota(jnp.int32, (blk,), 0)
        d_ref[...] = jnp.where(gi < e, ei_ref[1, :], n + gi % trash)

    return pl.pallas_call(
        body,
        grid=(grid,),
        in_specs=[pl.BlockSpec((2, blk), lambda b: (0, b))],
        out_specs=pl.BlockSpec((blk,), lambda b: (b,)),
        out_shape=jax.ShapeDtypeStruct((ep,), jnp.int32),
    )(edge_index)


def _tc_edge_prep_src(edge_index, n, ep):
    e = edge_index.shape[1]
    blk = 32768
    grid = ep // blk

    def body(ei_ref, s_ref, s2_ref):
        gi = pl.program_id(0) * blk + lax.broadcasted_iota(jnp.int32, (blk,), 0)
        s = jnp.where(gi < e, ei_ref[0, :], gi % n)
        s_ref[...] = s
        s2_ref[...] = 2 * s

    return pl.pallas_call(
        body,
        grid=(grid,),
        in_specs=[pl.BlockSpec((2, blk), lambda b: (0, b))],
        out_specs=[pl.BlockSpec((blk,), lambda b: (b,))] * 2,
        out_shape=[jax.ShapeDtypeStruct((ep,), jnp.int32)] * 2,
    )(edge_index)


def _sc_degree(dst2d, zeros16, ones16, n_pad, nch_w):
    rows_w = n_pad // _NSUB
    ngrp = rows_w // 16
    npair_w = rows_w // 2

    def body(dst_hbm, zeros_hbm, ones_hbm, outv_hbm, oute_hbm,
             didx_all, ones_v, cnt_v, deg_v, dege_v, acc, ssem):
        cid = lax.axis_index("c")
        sid = lax.axis_index("s")
        wid = cid * _NSUB + sid
        pltpu.sync_copy(zeros_hbm, acc.at[pl.ds(sid * rows_w, rows_w)])
        pltpu.sync_copy(ones_hbm, ones_v)
        pltpu.sync_copy(dst_hbm.at[pl.ds(wid * nch_w, nch_w)], didx_all)
        plsc.subcore_barrier()

        def fire(ci, _):
            pltpu.async_copy(ones_v, acc.at[didx_all.at[ci]], ssem, add=True)
            return ()

        def drain(ci, _):
            pltpu.make_async_copy(ones_v, acc.at[didx_all.at[ci]], ssem).wait()
            return ()

        lax.fori_loop(0, nch_w, fire, ())
        lax.fori_loop(0, nch_w, drain, ())
        plsc.subcore_barrier()

        pltpu.sync_copy(acc.at[pl.ds(sid * rows_w, rows_w)], cnt_v)
        riota = lax.iota(jnp.int32, 16)
        zidx = jnp.zeros((16,), jnp.int32)

        def compress(g, _):
            vals = plsc.load_gather(cnt_v, [g * 16 + riota, zidx])
            deg_v[pl.ds(g * 16, 16)] = vals
            return ()

        lax.fori_loop(0, ngrp, compress, ())
        pltpu.sync_copy(deg_v, outv_hbm.at[cid, pl.ds(sid * rows_w, rows_w)])

        def expand(r, _):
            v0 = cnt_v[2 * r, :]
            v1 = cnt_v[2 * r + 1, :]
            for k in range(4):
                dege_v[r, pl.ds(16 * k, 16)] = v0
            for k in range(4, 8):
                dege_v[r, pl.ds(16 * k, 16)] = v1
            return ()

        lax.fori_loop(0, npair_w, expand, ())
        pltpu.sync_copy(dege_v, oute_hbm.at[cid, pl.ds(sid * npair_w, npair_w)])

    fn = pl.kernel(
        body,
        out_type=[jax.ShapeDtypeStruct((_NSC, n_pad), jnp.float32),
                  jax.ShapeDtypeStruct((_NSC, n_pad // 2, 128), jnp.float32)],
        mesh=_sc_mesh(),
        compiler_params=pltpu.CompilerParams(use_tc_tiling_on_sc=False,
                                             needs_layout_passes=False),
        scratch_types=[
            pltpu.VMEM((nch_w, _LANES), jnp.int32),
            pltpu.VMEM((_LANES, 16), jnp.float32),
            pltpu.VMEM((rows_w, 16), jnp.float32),
            pltpu.VMEM((rows_w,), jnp.float32),
            pltpu.VMEM((npair_w, 128), jnp.float32),
            pltpu.VMEM_SHARED((n_pad, 16), jnp.float32),
            pltpu.SemaphoreType.DMA,
        ],
    )
    return fn(dst2d, zeros16, ones16)


def _sc_spmm(y, doubled_idx, src2d, dst2d, zeros_f, n, n_pad, nch_w):
    f = y.shape[1]
    rows_w = n_pad // _NSUB
    npairs = nch_w // 2
    nself = rows_w // _LANES
    full_tiles = n // rows_w
    rem = n % rows_w

    def body(y_hbm, src_hbm, dst_hbm, zeros_hbm, out_hbm,
             sidx_all, didx_all, sidx_self, didx_self, rows0, rows1, acc,
             gsem0, gsem1):
        cid = lax.axis_index("c")
        sid = lax.axis_index("s")
        wid = cid * _NSUB + sid

        if doubled_idx:
            pltpu.sync_copy(zeros_hbm, acc.at[pl.ds(sid * rows_w, rows_w)])
            riota = lax.iota(jnp.int32, 16)
            base_node = sid * rows_w
            for c in range(nself):
                for g in range(8):
                    nodes = base_node + (c * 128 + g * 16) + riota
                    didx_self[c, pl.ds(16 * g, 16)] = nodes
                    sidx_self[c, pl.ds(16 * g, 16)] = (
                        jnp.minimum(nodes, n - 1) * 2)
        else:
            @pl.when(jnp.logical_and(cid == 0, sid < full_tiles))
            def _():
                pltpu.sync_copy(y_hbm.at[pl.ds(sid * rows_w, rows_w)],
                                acc.at[pl.ds(sid * rows_w, rows_w)])

            @pl.when(jnp.logical_and(cid == 0, sid >= full_tiles))
            def _():
                if rem:
                    pltpu.sync_copy(y_hbm.at[pl.ds(sid * rows_w, rem)],
                                    acc.at[pl.ds(sid * rows_w, rem)])
                pltpu.sync_copy(
                    zeros_hbm.at[pl.ds(0, rows_w - rem)],
                    acc.at[pl.ds(sid * rows_w + rem, rows_w - rem)])

            @pl.when(cid != 0)
            def _():
                pltpu.sync_copy(zeros_hbm,
                                acc.at[pl.ds(sid * rows_w, rows_w)])

        pltpu.sync_copy(src_hbm.at[pl.ds(wid * nch_w, nch_w)], sidx_all)
        pltpu.sync_copy(dst_hbm.at[pl.ds(wid * nch_w, nch_w)], didx_all)
        plsc.subcore_barrier()

        pltpu.async_copy(y_hbm.at[sidx_all.at[0]], rows0, gsem0)

        def step(i, _):
            c0 = 2 * i
            c1 = c0 + 1
            pltpu.async_copy(y_hbm.at[sidx_all.at[c1]], rows1, gsem1)
            pltpu.make_async_copy(y_hbm.at[sidx_all.at[c0]], rows0, gsem0).wait()
            pltpu.sync_copy(rows0, acc.at[didx_all.at[c0]], add=True)
            cn = jnp.minimum(c0 + 2, nch_w - 1)
            pltpu.async_copy(y_hbm.at[sidx_all.at[cn]], rows0, gsem0)
            pltpu.make_async_copy(y_hbm.at[sidx_all.at[c1]], rows1, gsem1).wait()
            pltpu.sync_copy(rows1, acc.at[didx_all.at[c1]], add=True)
            return ()

        lax.fori_loop(0, npairs, step, ())
        pltpu.make_async_copy(y_hbm.at[sidx_all.at[nch_w - 1]], rows0,
                              gsem0).wait()
        if doubled_idx:
            for parity in range(2):
                lst = list(range(parity, nself, 2))

                @pl.when(cid == parity)
                def _(lst=lst):
                    bufs = ((rows0, gsem0), (rows1, gsem1))
                    for j in range(min(2, len(lst))):
                        pltpu.async_copy(y_hbm.at[sidx_self.at[lst[j]]],
                                         bufs[j][0], bufs[j][1])
                    for j, c in enumerate(lst):
                        buf, sem = bufs[j % 2]
                        pltpu.make_async_copy(y_hbm.at[sidx_self.at[c]],
                                              buf, sem).wait()
                        pltpu.sync_copy(buf, acc.at[didx_self.at[c]],
                                        add=True)
                        if j + 2 < len(lst):
                            pltpu.async_copy(
                                y_hbm.at[sidx_self.at[lst[j + 2]]], buf, sem)
        plsc.subcore_barrier()
        pltpu.sync_copy(acc.at[pl.ds(sid * rows_w, rows_w)],
                        out_hbm.at[cid, pl.ds(sid * rows_w, rows_w)])

    fn = pl.kernel(
        body,
        out_type=jax.ShapeDtypeStruct((_NSC, n_pad, f), jnp.float32),
        mesh=_sc_mesh(),
        compiler_params=pltpu.CompilerParams(use_tc_tiling_on_sc=False),
        scratch_types=[
            pltpu.VMEM((nch_w, _LANES), jnp.int32),
            pltpu.VMEM((nch_w, _LANES), jnp.int32),
            pltpu.VMEM((nself, _LANES), jnp.int32),
            pltpu.VMEM((nself, _LANES), jnp.int32),
            pltpu.VMEM((_LANES, f), jnp.float32),
            pltpu.VMEM((_LANES, f), jnp.float32),
            pltpu.VMEM_SHARED((n_pad, f), jnp.float32),
            pltpu.SemaphoreType.DMA,
            pltpu.SemaphoreType.DMA,
        ],
    )
    return fn(y, src2d, dst2d, zeros_f)


def _tc_scale_first(degv128, x, W, n):
    c_out = W.shape[1]
    xb = 1024
    grid = _cdiv(n, xb)

    def body(degv_ref, x_ref, w_ref, z_ref):
        pid = pl.program_id(0)
        nrow = xb // 128
        deg = (degv_ref[0, pl.ds(nrow * pid, nrow), :]
               + degv_ref[1, pl.ds(nrow * pid, nrow), :]) + 1.0
        dis_t = lax.transpose(lax.rsqrt(deg), (1, 0))
        dcol = jnp.concatenate(
            [dis_t[:, k:k + 1] for k in range(nrow)], axis=0)
        xw = jnp.dot(x_ref[...], w_ref[...],
                     preferred_element_type=jnp.float32)
        z_ref[...] = jnp.concatenate(
            [xw * dcol, jnp.zeros((xb, 128 - c_out), jnp.float32)], axis=1)

    return pl.pallas_call(
        body,
        grid=(grid,),
        in_specs=[
            pl.BlockSpec(degv128.shape, lambda b: (0, 0, 0)),
            pl.BlockSpec((xb, x.shape[1]), lambda b: (b, 0)),
            pl.BlockSpec((x.shape[1], c_out), lambda b: (0, 0)),
        ],
        out_specs=pl.BlockSpec((xb, 128), lambda b: (b, 0)),
        out_shape=jax.ShapeDtypeStruct((n, 128), jnp.float32),
    )(degv128, x, W)


def _tc_mid(up128, degE, n2):
    def body(up_ref, de_ref, v_ref):
        u = up_ref[0, :n2, :] + up_ref[1, :n2, :]
        deg = de_ref[0, :n2, :] + de_ref[1, :n2, :] + 1.0
        v_ref[...] = u / deg

    return pl.pallas_call(
        body,
        out_shape=jax.ShapeDtypeStruct((n2, 128), jnp.float32),
    )(up128, degE)


def _tc_final(wp128, degE, b2, n2):
    c_out = b2.shape[1] // 2

    def body(wp_ref, de_ref, b_ref, o_ref):
        w = wp_ref[0, :n2, :] + wp_ref[1, :n2, :]
        deg = de_ref[0, :n2, :] + de_ref[1, :n2, :] + 1.0
        logits = w * lax.rsqrt(deg) + b_ref[...]

        def lsm(l):
            m = jnp.max(l, axis=-1, keepdims=True)
            ex = jnp.exp(l - m)
            return l - (jnp.log(jnp.sum(ex, axis=-1, keepdims=True)) + m)

        o_ref[...] = jnp.concatenate(
            [lsm(logits[:, :c_out]), lsm(logits[:, c_out:])], axis=1)

    return pl.pallas_call(
        body,
        out_shape=jax.ShapeDtypeStruct((n2, 2 * c_out), jnp.float32),
    )(wp128, degE, b2)


def kernel(x, edge_index, W, b):
    n = x.shape[0]
    c_out = W.shape[1]
    e = edge_index.shape[1]
    n2 = n // 2

    n_pad = _cdiv(n + 1, 8 * _LANES) * 8 * _LANES
    nch = _cdiv(e, _LANES * _NW * 8) * _NW * 8
    nch_w = nch // _NW
    ep = nch * _LANES

    dst_f = _tc_edge_prep_dst(edge_index, n, n_pad, ep)
    src_f, src2_f = _tc_edge_prep_src(edge_index, n, ep)
    src_p = src_f.reshape(nch, _LANES)
    src2_p = src2_f.reshape(nch, _LANES)
    dst_p = dst_f.reshape(nch, _LANES)

    rows_w = n_pad // _NSUB
    zeros16 = jnp.zeros((rows_w, 16), jnp.float32)
    zerosf = jnp.zeros((rows_w, c_out), jnp.float32)
    ones16 = jnp.ones((_LANES, 16), jnp.float32)
    b2 = jnp.concatenate([b, b]).reshape(1, 2 * c_out)

    degv, degE = _sc_degree(dst_p, zeros16, ones16, n_pad, nch_w)
    degv128 = degv.reshape(_NSC, n_pad // _LANES, _LANES)
    zwide = _tc_scale_first(degv128, x, W, n)
    up = _sc_spmm(zwide.reshape(2 * n, c_out), True,
                  src2_p, dst_p, zerosf, n, n_pad, nch_w)
    v128 = _tc_mid(up.reshape(_NSC, n_pad // 2, 2 * c_out), degE, n2)
    wp = _sc_spmm(v128.reshape(n, c_out), False,
                  src_p, dst_p, zerosf, n, n_pad, nch_w)
    out128 = _tc_final(wp.reshape(_NSC, n_pad // 2, 2 * c_out), degE, b2, n2)
    return out128.reshape(n, c_out)
